# baseline (speedup 1.0000x reference)
import jax
import jax.numpy as jnp
from jax import lax
from jax.experimental import pallas as pl
from jax.experimental.pallas import tpu as pltpu
from jax.experimental.pallas import tpu_sc as plsc

N = 1024
NC, NS, L = 1, 16, 16
NW = NC * NS

N1 = 2048
P1 = (2 * N1) // NW
N0 = 1023
P0 = 2048 // NW
P0L = 2 * N0 - (NW - 1) * P0
V0C = 3 * P0 // 2
V0L = 3 * N0 - (NW - 1) * V0C
GCH = 128
MASK = N * N - 1

V1OFF = N * N
V0OFF = V1OFF + 4 * N1
O0OFF = 2 * N1


def _body(buf, o1, o0, vm1, vm0, idx1, idx0, g1, g0, sem):
    wid = lax.axis_index("c") * NS + lax.axis_index("s")
    last = wid == NW - 1
    lanes = jnp.arange(L, dtype=jnp.int32)

    ci1 = pltpu.async_copy(
        buf.at[pl.ds(V1OFF + wid * 2 * P1, 2 * P1)], vm1, sem)

    @pl.when(jnp.logical_not(last))
    def _():
        pltpu.sync_copy(buf.at[pl.ds(V0OFF + wid * V0C, V0C)], vm0)

    @pl.when(last)
    def _():
        pltpu.sync_copy(buf.at[pl.ds(V0OFF + (NW - 1) * V0C, V0L)],
                        vm0.at[pl.ds(0, V0L)])

    ci1.wait()

    for g in range(P1 // L):
        p = lanes + (g * L)
        a = 2 * p
        va = plsc.load_gather(vm1, [a])
        vb = plsc.load_gather(vm1, [a + 1])
        idx1[pl.ds(g * L, L)] = va * N + vb

    for g in range(P0 // L):
        p = lanes + (g * L)
        i3 = 3 * (p >> 1)
        par = p & 1
        va = plsc.load_gather(vm0, [i3 + par])
        vb = plsc.load_gather(vm0, [i3 + 2 * par])
        idx0[pl.ds(g * L, L)] = (va * N + vb) & MASK

    cps = [
        pltpu.async_copy(buf.at[idx1.at[pl.ds(c * GCH, GCH)]],
                         g1.at[pl.ds(c * GCH, GCH)], sem)
        for c in range(P1 // GCH)
    ] + [
        pltpu.async_copy(buf.at[idx0.at[pl.ds(c * GCH, GCH)]],
                         g0.at[pl.ds(c * GCH, GCH)], sem)
        for c in range(P0 // GCH)
    ]
    for c in cps:
        c.wait()

    co1 = pltpu.async_copy(g1, o1.at[pl.ds(wid * P1, P1)], sem)

    @pl.when(jnp.logical_not(last))
    def _():
        pltpu.sync_copy(g0, o0.at[pl.ds(wid * P0, P0)])

    @pl.when(last)
    def _():
        pltpu.sync_copy(g0.at[pl.ds(0, P0L)],
                        o0.at[pl.ds((NW - 1) * P0, P0L)])

    co1.wait()


def kernel(input, verts0, verts1):
    f32 = jnp.float32
    i32 = jnp.int32
    buf = jnp.concatenate([
        jax.lax.bitcast_convert_type(input, i32).reshape(-1),
        verts1.astype(i32).reshape(-1),
        verts0.astype(i32).reshape(-1),
    ])

    mesh = plsc.VectorSubcoreMesh(
        core_axis_name="c", subcore_axis_name="s", num_cores=NC)
    o1, o0 = pl.kernel(
        _body,
        out_type=(
            jax.ShapeDtypeStruct((2 * N1,), jnp.int32),
            jax.ShapeDtypeStruct((2 * N0,), jnp.int32),
        ),
        mesh=mesh,
        compiler_params=pltpu.CompilerParams(
            needs_layout_passes=False,
            skip_device_barrier=True,
            disable_bounds_checks=True,
            disable_semaphore_checks=True,
        ),
        scratch_types=[
            pltpu.VMEM((2 * P1,), jnp.int32),
            pltpu.VMEM((V0C,), jnp.int32),
            pltpu.VMEM((P1,), jnp.int32),
            pltpu.VMEM((P0,), jnp.int32),
            pltpu.VMEM((P1,), jnp.int32),
            pltpu.VMEM((P0,), jnp.int32),
            pltpu.SemaphoreType.DMA,
        ],
    )(buf)

    dgm0 = jax.lax.bitcast_convert_type(o0, f32).reshape(N0, 2)
    dgm1 = jax.lax.bitcast_convert_type(o1, f32).reshape(N1, 2)
    return (dgm0, dgm1)


# speedup vs baseline: 1.0990x; 1.0990x over previous
import jax
import jax.numpy as jnp
from jax import lax
from jax.experimental import pallas as pl
from jax.experimental.pallas import tpu as pltpu
from jax.experimental.pallas import tpu_sc as plsc

N = 1024
NC, NS, L = 1, 16, 16
NW = NC * NS

N1 = 2048
P1 = (2 * N1) // NW
N0 = 1023
P0 = 2048 // NW
P0L = 2 * N0 - (NW - 1) * P0
V0C = 3 * P0 // 2
V0L = 3 * N0 - (NW - 1) * V0C
GCH = 128
MASK = N * N - 1

V0OFF = 4 * N1


def _body(dist, vv, o1, o0, vm1, vm0, idx1, idx0, g1, g0, sem):
    wid = lax.axis_index("c") * NS + lax.axis_index("s")
    last = wid == NW - 1
    lanes = jnp.arange(L, dtype=jnp.int32)

    ci1 = pltpu.async_copy(
        vv.at[pl.ds(wid * 2 * P1, 2 * P1)], vm1, sem)

    @pl.when(jnp.logical_not(last))
    def _():
        pltpu.sync_copy(vv.at[pl.ds(V0OFF + wid * V0C, V0C)], vm0)

    @pl.when(last)
    def _():
        pltpu.sync_copy(vv.at[pl.ds(V0OFF + (NW - 1) * V0C, V0L)],
                        vm0.at[pl.ds(0, V0L)])

    ci1.wait()

    for g in range(P1 // L):
        p = lanes + (g * L)
        a = 2 * p
        va = plsc.load_gather(vm1, [a])
        vb = plsc.load_gather(vm1, [a + 1])
        idx1[pl.ds(g * L, L)] = va * N + vb

    for g in range(P0 // L):
        p = lanes + (g * L)
        i3 = 3 * (p >> 1)
        par = p & 1
        va = plsc.load_gather(vm0, [i3 + par])
        vb = plsc.load_gather(vm0, [i3 + 2 * par])
        idx0[pl.ds(g * L, L)] = (va * N + vb) & MASK

    cps = [
        pltpu.async_copy(dist.at[idx1.at[pl.ds(c * GCH, GCH)]],
                         g1.at[pl.ds(c * GCH, GCH)], sem)
        for c in range(P1 // GCH)
    ] + [
        pltpu.async_copy(dist.at[idx0.at[pl.ds(c * GCH, GCH)]],
                         g0.at[pl.ds(c * GCH, GCH)], sem)
        for c in range(P0 // GCH)
    ]
    for c in cps:
        c.wait()

    co1 = pltpu.async_copy(g1, o1.at[pl.ds(wid * P1, P1)], sem)

    @pl.when(jnp.logical_not(last))
    def _():
        pltpu.sync_copy(g0, o0.at[pl.ds(wid * P0, P0)])

    @pl.when(last)
    def _():
        pltpu.sync_copy(g0.at[pl.ds(0, P0L)],
                        o0.at[pl.ds((NW - 1) * P0, P0L)])

    co1.wait()


def kernel(input, verts0, verts1):
    f32 = jnp.float32
    i32 = jnp.int32
    dist = input.reshape(-1)
    vv = jnp.concatenate([
        verts1.astype(i32).reshape(-1),
        verts0.astype(i32).reshape(-1),
    ])

    mesh = plsc.VectorSubcoreMesh(
        core_axis_name="c", subcore_axis_name="s", num_cores=NC)
    o1, o0 = pl.kernel(
        _body,
        out_type=(
            jax.ShapeDtypeStruct((2 * N1,), f32),
            jax.ShapeDtypeStruct((2 * N0,), f32),
        ),
        mesh=mesh,
        compiler_params=pltpu.CompilerParams(
            needs_layout_passes=False,
            skip_device_barrier=True,
            disable_bounds_checks=True,
            disable_semaphore_checks=True,
        ),
        scratch_types=[
            pltpu.VMEM((2 * P1,), jnp.int32),
            pltpu.VMEM((V0C,), jnp.int32),
            pltpu.VMEM((P1,), jnp.int32),
            pltpu.VMEM((P0,), jnp.int32),
            pltpu.VMEM((P1,), f32),
            pltpu.VMEM((P0,), f32),
            pltpu.SemaphoreType.DMA,
        ],
    )(dist, vv)

    dgm0 = o0.reshape(N0, 2)
    dgm1 = o1.reshape(N1, 2)
    return (dgm0, dgm1)


# speedup vs baseline: 1.1054x; 1.0058x over previous
import jax
import jax.numpy as jnp
from jax import lax
from jax.experimental import pallas as pl
from jax.experimental.pallas import tpu as pltpu
from jax.experimental.pallas import tpu_sc as plsc

N = 1024
NC, NS, L = 1, 16, 16
NW = NC * NS

N1 = 2048
P1 = (2 * N1) // NW
N0 = 1023
P0 = 2048 // NW
P0L = 2 * N0 - (NW - 1) * P0
V0C = 3 * P0 // 2
V0L = 3 * N0 - (NW - 1) * V0C
GCH = 128
MASK = N * N - 1

V0OFF = 4 * N1


def _body(dist, vv, o1, o0, vm1, vm0, idx1, idx0, g1, g0, sem):
    wid = lax.axis_index("c") * NS + lax.axis_index("s")
    last = wid == NW - 1
    lanes = jnp.arange(L, dtype=jnp.int32)

    ci1 = pltpu.async_copy(
        vv.at[pl.ds(wid * 2 * P1, 2 * P1)], vm1, sem)

    @pl.when(jnp.logical_not(last))
    def _():
        pltpu.sync_copy(vv.at[pl.ds(V0OFF + wid * V0C, V0C)], vm0)

    @pl.when(last)
    def _():
        pltpu.sync_copy(vv.at[pl.ds(V0OFF + (NW - 1) * V0C, V0L)],
                        vm0.at[pl.ds(0, V0L)])

    ci1.wait()

    def ph1(g, _):
        p = lanes + g * L
        a = 2 * p
        va = plsc.load_gather(vm1, [a])
        vb = plsc.load_gather(vm1, [a + 1])
        idx1[pl.ds(g * L, L)] = va * N + vb
        return 0

    lax.fori_loop(0, P1 // L, ph1, 0, unroll=2)

    def ph0(g, _):
        p = lanes + g * L
        i3 = 3 * (p >> 1)
        par = p & 1
        va = plsc.load_gather(vm0, [i3 + par])
        vb = plsc.load_gather(vm0, [i3 + 2 * par])
        idx0[pl.ds(g * L, L)] = (va * N + vb) & MASK
        return 0

    lax.fori_loop(0, P0 // L, ph0, 0, unroll=2)

    cps = [
        pltpu.async_copy(dist.at[idx1.at[pl.ds(c * GCH, GCH)]],
                         g1.at[pl.ds(c * GCH, GCH)], sem)
        for c in range(P1 // GCH)
    ] + [
        pltpu.async_copy(dist.at[idx0.at[pl.ds(c * GCH, GCH)]],
                         g0.at[pl.ds(c * GCH, GCH)], sem)
        for c in range(P0 // GCH)
    ]
    for c in cps:
        c.wait()

    co1 = pltpu.async_copy(g1, o1.at[pl.ds(wid * P1, P1)], sem)

    @pl.when(jnp.logical_not(last))
    def _():
        pltpu.sync_copy(g0, o0.at[pl.ds(wid * P0, P0)])

    @pl.when(last)
    def _():
        pltpu.sync_copy(g0.at[pl.ds(0, P0L)],
                        o0.at[pl.ds((NW - 1) * P0, P0L)])

    co1.wait()


def kernel(input, verts0, verts1):
    f32 = jnp.float32
    i32 = jnp.int32
    dist = input.reshape(-1)
    vv = jnp.concatenate([
        verts1.astype(i32).reshape(-1),
        verts0.astype(i32).reshape(-1),
    ])

    mesh = plsc.VectorSubcoreMesh(
        core_axis_name="c", subcore_axis_name="s", num_cores=NC)
    o1, o0 = pl.kernel(
        _body,
        out_type=(
            jax.ShapeDtypeStruct((2 * N1,), f32),
            jax.ShapeDtypeStruct((2 * N0,), f32),
        ),
        mesh=mesh,
        compiler_params=pltpu.CompilerParams(
            needs_layout_passes=False,
            skip_device_barrier=True,
            disable_bounds_checks=True,
            disable_semaphore_checks=True,
        ),
        scratch_types=[
            pltpu.VMEM((2 * P1,), jnp.int32),
            pltpu.VMEM((V0C,), jnp.int32),
            pltpu.VMEM((P1,), jnp.int32),
            pltpu.VMEM((P0,), jnp.int32),
            pltpu.VMEM((P1,), f32),
            pltpu.VMEM((P0,), f32),
            pltpu.SemaphoreType.DMA,
        ],
    )(dist, vv)

    dgm0 = o0.reshape(N0, 2)
    dgm1 = o1.reshape(N1, 2)
    return (dgm0, dgm1)


# speedup vs baseline: 1.1065x; 1.0011x over previous
import jax
import jax.numpy as jnp
from jax import lax
from jax.experimental import pallas as pl
from jax.experimental.pallas import tpu as pltpu
from jax.experimental.pallas import tpu_sc as plsc

N = 1024
NC, NS, L = 1, 16, 16
NW = NC * NS

N1 = 2048
P1 = (2 * N1) // NW
N0 = 1023
P0 = 2048 // NW
P0L = 2 * N0 - (NW - 1) * P0
V0C = 3 * P0 // 2
V0L = 3 * N0 - (NW - 1) * V0C
GCH = 128
MASK = N * N - 1

V0OFF = 4 * N1


def _body(dist, vv, o1, o0, vm1, vm0, idx1, idx0, g1, g0, sem):
    wid = lax.axis_index("c") * NS + lax.axis_index("s")
    last = wid == NW - 1
    lanes = jnp.arange(L, dtype=jnp.int32)

    ci1 = pltpu.async_copy(
        vv.at[pl.ds(wid * 2 * P1, 2 * P1)], vm1, sem)

    @pl.when(jnp.logical_not(last))
    def _():
        pltpu.sync_copy(vv.at[pl.ds(V0OFF + wid * V0C, V0C)], vm0)

    @pl.when(last)
    def _():
        pltpu.sync_copy(vv.at[pl.ds(V0OFF + (NW - 1) * V0C, V0L)],
                        vm0.at[pl.ds(0, V0L)])

    ci1.wait()

    def ph1(g, _):
        p = lanes + g * L
        a = 2 * p
        va = plsc.load_gather(vm1, [a])
        vb = plsc.load_gather(vm1, [a + 1])
        idx1[pl.ds(g * L, L)] = va * N + vb
        return 0

    lax.fori_loop(0, P1 // L, ph1, 0, unroll=2)

    def ph0(g, _):
        p = lanes + g * L
        i3 = 3 * (p >> 1)
        par = p & 1
        va = plsc.load_gather(vm0, [i3 + par])
        vb = plsc.load_gather(vm0, [i3 + 2 * par])
        idx0[pl.ds(g * L, L)] = (va * N + vb) & MASK
        return 0

    lax.fori_loop(0, P0 // L, ph0, 0, unroll=2)

    cps = [
        pltpu.async_copy(dist.at[idx1.at[pl.ds(c * GCH, GCH)]],
                         g1.at[pl.ds(c * GCH, GCH)], sem)
        for c in range(P1 // GCH)
    ] + [
        pltpu.async_copy(dist.at[idx0.at[pl.ds(c * GCH, GCH)]],
                         g0.at[pl.ds(c * GCH, GCH)], sem)
        for c in range(P0 // GCH)
    ]
    for c in cps:
        c.wait()

    co1 = pltpu.async_copy(g1, o1.at[pl.ds(wid * P1, P1)], sem)

    @pl.when(jnp.logical_not(last))
    def _():
        pltpu.sync_copy(g0, o0.at[pl.ds(wid * P0, P0)])

    @pl.when(last)
    def _():
        pltpu.sync_copy(g0.at[pl.ds(0, P0L)],
                        o0.at[pl.ds((NW - 1) * P0, P0L)])

    co1.wait()


def kernel(input, verts0, verts1):
    f32 = jnp.float32
    i32 = jnp.int32
    dist = input.reshape(-1)
    vv = jnp.concatenate([
        verts1.reshape(-1).astype(i32),
        verts0.reshape(-1).astype(i32),
    ])

    mesh = plsc.VectorSubcoreMesh(
        core_axis_name="c", subcore_axis_name="s", num_cores=NC)
    o1, o0 = pl.kernel(
        _body,
        out_type=(
            jax.ShapeDtypeStruct((2 * N1,), f32),
            jax.ShapeDtypeStruct((2 * N0,), f32),
        ),
        mesh=mesh,
        compiler_params=pltpu.CompilerParams(
            needs_layout_passes=False,
            skip_device_barrier=True,
            disable_bounds_checks=True,
            disable_semaphore_checks=True,
        ),
        scratch_types=[
            pltpu.VMEM((2 * P1,), jnp.int32),
            pltpu.VMEM((V0C,), jnp.int32),
            pltpu.VMEM((P1,), jnp.int32),
            pltpu.VMEM((P0,), jnp.int32),
            pltpu.VMEM((P1,), f32),
            pltpu.VMEM((P0,), f32),
            pltpu.SemaphoreType.DMA,
        ],
    )(dist, vv)

    dgm0 = o0.reshape(N0, 2)
    dgm1 = o1.reshape(N1, 2)
    return (dgm0, dgm1)
